# baseline (speedup 1.0000x reference)
import functools

import jax
import jax.numpy as jnp
from jax import lax
from jax.experimental import pallas as pl
from jax.experimental.pallas import tpu as pltpu
from jax.experimental.pallas import tpu_sc as plsc

NUM_EXPERTS = 8
HIDDEN = 1024
FFN = 2048
T = 2048
P = 2 * T
BLK = 128
NB = P // BLK + NUM_EXPERTS
PADDED = NB * BLK
NMETA = 64

NC, NS, L = 2, 16, 16
NW = NC * NS
ROWS_PER_W = PADDED // NW
LOCBUF = ROWS_PER_W + L
GCHUNK = 64
TOK_PER_W = T // NW
CCHUNK = 32


def _router_body(h_ref, gate_ref, pos_ref, wp_ref, meta_ref):
    h = h_ref[...]
    logits = jax.lax.dot_general(
        gate_ref[...], h, (((1,), (1,)), ((), ())),
        preferred_element_type=jnp.float32)
    m = jnp.max(logits, axis=0, keepdims=True)
    ex = jnp.exp(logits - m)
    p = ex / jnp.sum(ex, axis=0, keepdims=True)
    eidx = jax.lax.broadcasted_iota(jnp.int32, p.shape, 0)
    BIG = jnp.int32(NUM_EXPERTS)
    m1 = jnp.max(p, axis=0, keepdims=True)
    i1 = jnp.min(jnp.where(p == m1, eidx, BIG), axis=0, keepdims=True)
    mask1 = eidx == i1
    p2 = jnp.where(mask1, -1.0, p)
    m2 = jnp.max(p2, axis=0, keepdims=True)
    i2 = jnp.min(jnp.where(p2 == m2, eidx, BIG), axis=0, keepdims=True)
    mask2 = eidx == i2
    denom = m1 + m2
    wp_ref[0:1, :] = m1 / denom
    wp_ref[1:2, :] = m2 / denom

    tri = (jax.lax.broadcasted_iota(jnp.int32, (T, T), 0)
           <= jax.lax.broadcasted_iota(jnp.int32, (T, T), 1)
           ).astype(jnp.float32)
    pos0 = jnp.zeros((1, T), jnp.float32)
    pos1 = jnp.zeros((1, T), jnp.float32)
    eo = jnp.float32(0.0)
    eo_list = []
    for e in range(NUM_EXPERTS):
        me0 = mask1[e:e + 1, :].astype(jnp.float32)
        me1 = mask2[e:e + 1, :].astype(jnp.float32)
        csum0 = jax.lax.dot_general(me0, tri, (((1,), (0,)), ((), ())),
                                    preferred_element_type=jnp.float32)
        csum1 = jax.lax.dot_general(me1, tri, (((1,), (0,)), ((), ())),
                                    preferred_element_type=jnp.float32)
        rs0 = jnp.sum(me0)
        eo_list.append(eo)
        pos0 = pos0 + me0 * (csum0 - 1.0 + eo)
        pos1 = pos1 + me1 * (csum1 - 1.0 + rs0 + eo)
        cnt = rs0 + jnp.sum(me1)
        cnt_i = cnt.astype(jnp.int32)
        pc = ((cnt_i + BLK - 1) // BLK) * BLK
        eo = eo + pc.astype(jnp.float32)
    pos_ref[0:1, :] = pos0.astype(jnp.int32)
    pos_ref[1:2, :] = pos1.astype(jnp.int32)

    bidx = (jax.lax.broadcasted_iota(jnp.int32, (1, NMETA), 1)
            .astype(jnp.float32) * float(BLK))
    be = jnp.zeros((1, NMETA), jnp.float32)
    for e in range(1, NUM_EXPERTS):
        be = be + (bidx >= eo_list[e]).astype(jnp.float32)
    nu = eo * (1.0 / BLK)
    lane = jax.lax.broadcasted_iota(jnp.int32, (1, NMETA), 1)
    meta = jnp.where(lane == NB, nu, be)
    meta_ref[...] = meta.astype(jnp.int32)


def _router(hidden, gate_w):
    return pl.pallas_call(
        _router_body,
        in_specs=[
            pl.BlockSpec((T, HIDDEN), lambda: (0, 0)),
            pl.BlockSpec((NUM_EXPERTS, HIDDEN), lambda: (0, 0)),
        ],
        out_specs=[
            pl.BlockSpec((2, T), lambda: (0, 0)),
            pl.BlockSpec((2, T), lambda: (0, 0)),
            pl.BlockSpec((1, NMETA), lambda: (0, 0)),
        ],
        out_shape=[
            jax.ShapeDtypeStruct((2, T), jnp.int32),
            jax.ShapeDtypeStruct((2, T), jnp.float32),
            jax.ShapeDtypeStruct((1, NMETA), jnp.int32),
        ],
    )(hidden, gate_w)


PAIRS_PER_W = P // NW
DCHUNK = 64
NDCHUNK = PAIRS_PER_W // DCHUNK


def _dispatch_sc_body(pos_hbm, wp_hbm, hidden_hbm, a_hbm, ws_hbm,
                      pos_v, wp_v, rows_v, sem, wsem):
    cid = lax.axis_index("c")
    sid = lax.axis_index("s")
    wid = sid * NC + cid
    r0 = (wid % (NW // 2)) * PAIRS_PER_W
    p0 = wid * PAIRS_PER_W
    pltpu.sync_copy(pos_hbm.at[wid], pos_v)
    for chunk in range(NDCHUNK):
        base = chunk * DCHUNK
        pltpu.sync_copy(wp_hbm.at[pl.ds(p0 + base, DCHUNK)],
                        wp_v.at[chunk])
        pltpu.async_copy(wp_v.at[chunk], ws_hbm.at[pos_v.at[chunk]],
                         wsem).wait()
        pltpu.sync_copy(hidden_hbm.at[pl.ds(r0 + base, DCHUNK)], rows_v)
        pltpu.async_copy(rows_v, a_hbm.at[pos_v.at[chunk]], sem).wait()


def _dispatch(pos3, wp_flat, hidden):
    mesh = plsc.VectorSubcoreMesh(core_axis_name="c", subcore_axis_name="s")
    fn = pl.kernel(
        _dispatch_sc_body, mesh=mesh,
        out_type=[
            jax.ShapeDtypeStruct((PADDED, HIDDEN), jnp.float32),
            jax.ShapeDtypeStruct((PADDED,), jnp.float32),
        ],
        scratch_types=[
            pltpu.VMEM((NDCHUNK, DCHUNK), jnp.int32),
            pltpu.VMEM((NDCHUNK, DCHUNK), jnp.float32),
            pltpu.VMEM((DCHUNK, HIDDEN), jnp.float32),
            pltpu.SemaphoreType.DMA,
            pltpu.SemaphoreType.DMA,
        ],
    )
    return fn(pos3, wp_flat, hidden)


def _silu(x):
    return x * (1.0 / (1.0 + jnp.exp(-x)))


def _ffn_body(m_ref, a_ref, ws_ref, wg_ref, wu_ref, wd_ref, out_ref):
    b = pl.program_id(0)
    nu = m_ref[NB]

    @pl.when(b < nu)
    def _():
        a = a_ref[...]
        g = jax.lax.dot_general(a, wg_ref[0], (((1,), (1,)), ((), ())),
                                preferred_element_type=jnp.float32)
        u = jax.lax.dot_general(a, wu_ref[0], (((1,), (1,)), ((), ())),
                                preferred_element_type=jnp.float32)
        act = _silu(g) * u
        o = jax.lax.dot_general(act, wd_ref[0], (((1,), (1,)), ((), ())),
                                preferred_element_type=jnp.float32)
        out_ref[...] = o * ws_ref[...]


def _ffn_gemm(meta, a_sorted, w_sorted_col, wg, wu, wd):
    def amap(b, m):
        bc = jnp.minimum(b, m[NB] - 1)
        return (bc, 0)

    def wmap(b, m):
        bc = jnp.minimum(b, m[NB] - 1)
        return (m[bc], 0, 0)

    return pl.pallas_call(
        _ffn_body,
        grid_spec=pltpu.PrefetchScalarGridSpec(
            num_scalar_prefetch=1,
            grid=(NB,),
            in_specs=[
                pl.BlockSpec((BLK, HIDDEN), amap),
                pl.BlockSpec((BLK, 1), amap),
                pl.BlockSpec((1, FFN, HIDDEN), wmap),
                pl.BlockSpec((1, FFN, HIDDEN), wmap),
                pl.BlockSpec((1, HIDDEN, FFN), wmap),
            ],
            out_specs=pl.BlockSpec((BLK, HIDDEN), lambda b, m: (b, 0)),
        ),
        out_shape=jax.ShapeDtypeStruct((PADDED, HIDDEN), jnp.float32),
        compiler_params=pltpu.CompilerParams(
            dimension_semantics=("arbitrary",),
        ),
    )(meta, a_sorted, w_sorted_col, wg, wu, wd)


def _combine_sc_body(wout_hbm, pos1_hbm, pos2_hbm, final_hbm,
                     idx1_v, idx2_v, r1_v, r2_v, sem):
    cid = lax.axis_index("c")
    sid = lax.axis_index("s")
    wid = sid * NC + cid
    base_t = wid * TOK_PER_W
    pltpu.sync_copy(pos1_hbm.at[pl.ds(base_t, TOK_PER_W)], idx1_v)
    pltpu.sync_copy(pos2_hbm.at[pl.ds(base_t, TOK_PER_W)], idx2_v)
    for chunk in range(TOK_PER_W // CCHUNK):
        cbase = chunk * CCHUNK
        pltpu.async_copy(
            wout_hbm.at[idx1_v.at[pl.ds(cbase, CCHUNK)]], r1_v, sem,
        ).wait()
        pltpu.async_copy(
            wout_hbm.at[idx2_v.at[pl.ds(cbase, CCHUNK)]], r2_v, sem,
        ).wait()

        def addbody(r, c):
            for j in range(HIDDEN // L):
                sl = pl.ds(j * L, L)
                r1_v[r, sl] = r1_v[r, sl] + r2_v[r, sl]
            return c
        lax.fori_loop(0, CCHUNK, addbody, 0)
        pltpu.sync_copy(r1_v, final_hbm.at[pl.ds(base_t + cbase, CCHUNK)])


def _combine(wout, pos1, pos2):
    mesh = plsc.VectorSubcoreMesh(core_axis_name="c", subcore_axis_name="s")
    fn = pl.kernel(
        _combine_sc_body, mesh=mesh,
        out_type=jax.ShapeDtypeStruct((T, HIDDEN), jnp.float32),
        scratch_types=[
            pltpu.VMEM((TOK_PER_W,), jnp.int32),
            pltpu.VMEM((TOK_PER_W,), jnp.int32),
            pltpu.VMEM((CCHUNK, HIDDEN), jnp.float32),
            pltpu.VMEM((CCHUNK, HIDDEN), jnp.float32),
            pltpu.SemaphoreType.DMA,
        ],
    )
    return fn(wout, pos1, pos2)


@jax.jit
def kernel(hidden_states, gate_w, wg, wu, wd):
    B, S, H = hidden_states.shape
    hidden = hidden_states.reshape(-1, H)
    pos, wp, meta = _router(hidden, gate_w)
    pos3 = pos.reshape(NW, NDCHUNK, DCHUNK)
    meta_flat = meta.reshape(NMETA)
    a_sorted, w_sorted = _dispatch(pos3, wp.reshape(P), hidden)
    wout = _ffn_gemm(meta_flat, a_sorted, w_sorted.reshape(PADDED, 1),
                     wg, wu, wd)
    final = _combine(wout, pos[0], pos[1])
    return final.reshape(B, S, H)


# speedup vs baseline: 1.3110x; 1.3110x over previous
import functools

import jax
import jax.numpy as jnp
from jax import lax
from jax.experimental import pallas as pl
from jax.experimental.pallas import tpu as pltpu
from jax.experimental.pallas import tpu_sc as plsc

NUM_EXPERTS = 8
HIDDEN = 1024
FFN = 2048
T = 2048
P = 2 * T
BLK = 256
NB = P // BLK + NUM_EXPERTS
PADDED = NB * BLK
F_BLK = 1024
NF = FFN // F_BLK
NMETA = 32

NC, NS, L = 2, 16, 16
NW = NC * NS
ROWS_PER_W = PADDED // NW
LOCBUF = ROWS_PER_W + L
GCHUNK = 64
TOK_PER_W = T // NW
CCHUNK = 16


def _router_body(h_ref, gate_ref, pos_ref, wp_ref, meta_ref):
    h = h_ref[...]
    logits = jax.lax.dot_general(
        gate_ref[...], h, (((1,), (1,)), ((), ())),
        preferred_element_type=jnp.float32)
    m = jnp.max(logits, axis=0, keepdims=True)
    ex = jnp.exp(logits - m)
    p = ex / jnp.sum(ex, axis=0, keepdims=True)
    eidx = jax.lax.broadcasted_iota(jnp.int32, p.shape, 0)
    BIG = jnp.int32(NUM_EXPERTS)
    m1 = jnp.max(p, axis=0, keepdims=True)
    i1 = jnp.min(jnp.where(p == m1, eidx, BIG), axis=0, keepdims=True)
    mask1 = eidx == i1
    p2 = jnp.where(mask1, -1.0, p)
    m2 = jnp.max(p2, axis=0, keepdims=True)
    i2 = jnp.min(jnp.where(p2 == m2, eidx, BIG), axis=0, keepdims=True)
    mask2 = eidx == i2
    denom = m1 + m2
    wp_ref[0:1, :] = m1 / denom
    wp_ref[1:2, :] = m2 / denom

    m1f = mask1.astype(jnp.float32)
    m2f = mask2.astype(jnp.float32)
    tri = (jax.lax.broadcasted_iota(jnp.int32, (T, T), 0)
           <= jax.lax.broadcasted_iota(jnp.int32, (T, T), 1)
           ).astype(jnp.float32)
    c1 = jax.lax.dot_general(m1f, tri, (((1,), (0,)), ((), ())),
                             preferred_element_type=jnp.float32)
    c2 = jax.lax.dot_general(m2f, tri, (((1,), (0,)), ((), ())),
                             preferred_element_type=jnp.float32)
    rs1 = jnp.sum(m1f, axis=1, keepdims=True)
    rs2 = jnp.sum(m2f, axis=1, keepdims=True)
    cnt = rs1 + rs2
    pc = jnp.floor((cnt + (BLK - 1.0)) * (1.0 / BLK)) * BLK
    s8 = (jax.lax.broadcasted_iota(jnp.int32, (NUM_EXPERTS, NUM_EXPERTS), 1)
          < jax.lax.broadcasted_iota(jnp.int32, (NUM_EXPERTS, NUM_EXPERTS), 0)
          ).astype(jnp.float32)
    eo = jax.lax.dot_general(s8, pc, (((1,), (0,)), ((), ())),
                             preferred_element_type=jnp.float32)
    pos0 = jnp.sum(m1f * (c1 - 1.0 + eo), axis=0, keepdims=True)
    pos1 = jnp.sum(m2f * (c2 - 1.0 + rs1 + eo), axis=0, keepdims=True)
    pos_ref[0:1, :] = pos0.astype(jnp.int32)
    pos_ref[1:2, :] = pos1.astype(jnp.int32)

    bidx = (jax.lax.broadcasted_iota(jnp.int32, (1, NMETA), 1)
            .astype(jnp.float32) * float(BLK))
    be = jnp.sum((bidx >= eo).astype(jnp.float32), axis=0,
                 keepdims=True) - 1.0
    nu = jnp.sum(pc) * (1.0 / BLK)
    lane = jax.lax.broadcasted_iota(jnp.int32, (1, NMETA), 1)
    meta = jnp.where(lane == NB, nu, be)
    meta_ref[...] = meta.astype(jnp.int32)


def _router(hidden, gate_w):
    return pl.pallas_call(
        _router_body,
        in_specs=[
            pl.BlockSpec((T, HIDDEN), lambda: (0, 0)),
            pl.BlockSpec((NUM_EXPERTS, HIDDEN), lambda: (0, 0)),
        ],
        out_specs=[
            pl.BlockSpec((2, T), lambda: (0, 0)),
            pl.BlockSpec((2, T), lambda: (0, 0)),
            pl.BlockSpec((1, NMETA), lambda: (0, 0)),
        ],
        out_shape=[
            jax.ShapeDtypeStruct((2, T), jnp.int32),
            jax.ShapeDtypeStruct((2, T), jnp.float32),
            jax.ShapeDtypeStruct((1, NMETA), jnp.int32),
        ],
    )(hidden, gate_w)


PAIRS_PER_W = P // NW
DCHUNK = 32
NDCHUNK = PAIRS_PER_W // DCHUNK


def _dispatch_sc_body(pos_hbm, wp_hbm, hidden_hbm, a_hbm, ws_hbm,
                      pos_v, wp_v, rows_v, sem, wsem):
    cid = lax.axis_index("c")
    sid = lax.axis_index("s")
    wid = sid * NC + cid
    r0 = (wid % (NW // 2)) * PAIRS_PER_W
    p0 = wid * PAIRS_PER_W
    pltpu.sync_copy(pos_hbm.at[wid], pos_v)
    pltpu.sync_copy(wp_hbm.at[wid], wp_v)
    wcopies = [pltpu.async_copy(wp_v.at[c], ws_hbm.at[pos_v.at[c]], wsem)
               for c in range(NDCHUNK)]
    copies = []
    for chunk in range(NDCHUNK):
        base = chunk * DCHUNK
        if chunk >= 2:
            copies[chunk - 2].wait()
        pltpu.sync_copy(hidden_hbm.at[pl.ds(r0 + base, DCHUNK)],
                        rows_v.at[chunk % 2])
        copies.append(pltpu.async_copy(rows_v.at[chunk % 2],
                                       a_hbm.at[pos_v.at[chunk]], sem))
    copies[NDCHUNK - 2].wait()
    copies[NDCHUNK - 1].wait()
    for c in wcopies:
        c.wait()


def _dispatch(pos3, wp_flat, hidden):
    mesh = plsc.VectorSubcoreMesh(core_axis_name="c", subcore_axis_name="s")
    fn = pl.kernel(
        _dispatch_sc_body, mesh=mesh,
        out_type=[
            jax.ShapeDtypeStruct((PADDED, HIDDEN), jnp.float32),
            jax.ShapeDtypeStruct((PADDED,), jnp.float32),
        ],
        scratch_types=[
            pltpu.VMEM((NDCHUNK, DCHUNK), jnp.int32),
            pltpu.VMEM((NDCHUNK, DCHUNK), jnp.float32),
            pltpu.VMEM((2, DCHUNK, HIDDEN), jnp.float32),
            pltpu.SemaphoreType.DMA,
            pltpu.SemaphoreType.DMA,
        ],
    )
    return fn(pos3, wp_flat, hidden)


def _silu(x):
    return x * (1.0 / (1.0 + jnp.exp(-x)))


def _ffn_body(m_ref, a_ref, ws_ref, wg_ref, wu_ref, wd_ref, out_ref):
    f = pl.program_id(0)
    b = pl.program_id(1)
    nu = m_ref[NB]

    @pl.when(b < nu)
    def _():
        a = a_ref[...]
        g = jax.lax.dot_general(a, wg_ref[0], (((1,), (1,)), ((), ())),
                                preferred_element_type=jnp.float32)
        u = jax.lax.dot_general(a, wu_ref[0], (((1,), (1,)), ((), ())),
                                preferred_element_type=jnp.float32)
        act = _silu(g) * u
        partial = jax.lax.dot_general(act, wd_ref[0], (((1,), (1,)), ((), ())),
                                      preferred_element_type=jnp.float32)
        rows = pl.ds(b * BLK, BLK)

        @pl.when(f == 0)
        def _():
            out_ref[rows, :] = partial

        @pl.when(f == NF - 1)
        def _():
            out_ref[rows, :] = (out_ref[rows, :] + partial) * ws_ref[...]


def _ffn_gemm(meta, a_sorted, w_sorted_col, wg, wu, wd):
    def amap(f, b, m):
        bc = jnp.minimum(b, m[NB] - 1)
        return (bc, 0)

    def wmap(f, b, m):
        bc = jnp.minimum(b, m[NB] - 1)
        return (m[bc], f, 0)

    def wdmap(f, b, m):
        bc = jnp.minimum(b, m[NB] - 1)
        return (m[bc], 0, f)

    return pl.pallas_call(
        _ffn_body,
        grid_spec=pltpu.PrefetchScalarGridSpec(
            num_scalar_prefetch=1,
            grid=(NF, NB),
            in_specs=[
                pl.BlockSpec((BLK, HIDDEN), amap),
                pl.BlockSpec((BLK, 1), amap),
                pl.BlockSpec((1, F_BLK, HIDDEN), wmap),
                pl.BlockSpec((1, F_BLK, HIDDEN), wmap),
                pl.BlockSpec((1, HIDDEN, F_BLK), wdmap),
            ],
            out_specs=pl.BlockSpec((PADDED, HIDDEN), lambda f, b, m: (0, 0)),
        ),
        out_shape=jax.ShapeDtypeStruct((PADDED, HIDDEN), jnp.float32),
        compiler_params=pltpu.CompilerParams(
            dimension_semantics=("arbitrary", "arbitrary"),
        ),
    )(meta, a_sorted, w_sorted_col, wg, wu, wd)


def _combine_sc_body(wout_hbm, pos1_hbm, pos2_hbm, final_hbm,
                     idx1_v, idx2_v, r1_v, r2_v, sem, sem2, sem3):
    cid = lax.axis_index("c")
    sid = lax.axis_index("s")
    wid = sid * NC + cid
    base_t = wid * TOK_PER_W
    NCC = TOK_PER_W // CCHUNK
    pltpu.sync_copy(pos1_hbm.at[pl.ds(base_t, TOK_PER_W)], idx1_v)
    pltpu.sync_copy(pos2_hbm.at[pl.ds(base_t, TOK_PER_W)], idx2_v)

    def gsrc(c, iv):
        return wout_hbm.at[iv.at[pl.ds(c * CCHUNK, CCHUNK)]]

    cur1 = pltpu.async_copy(gsrc(0, idx1_v), r1_v.at[0], sem)
    cur2 = pltpu.async_copy(gsrc(0, idx2_v), r2_v.at[0], sem2)
    stores = []
    for c in range(NCC):
        if c + 1 < NCC:
            if c >= 1:
                stores[c - 1].wait()
            nxt1 = pltpu.async_copy(gsrc(c + 1, idx1_v),
                                    r1_v.at[(c + 1) % 2], sem)
            nxt2 = pltpu.async_copy(gsrc(c + 1, idx2_v),
                                    r2_v.at[(c + 1) % 2], sem2)
        cur1.wait()
        cur2.wait()
        buf = c % 2

        def addbody(r, carry):
            for j in range(HIDDEN // L):
                sl = pl.ds(j * L, L)
                r1_v[buf, r, sl] = r1_v[buf, r, sl] + r2_v[buf, r, sl]
            return carry
        lax.fori_loop(0, CCHUNK, addbody, 0)
        stores.append(pltpu.async_copy(
            r1_v.at[buf],
            final_hbm.at[pl.ds(base_t + c * CCHUNK, CCHUNK)], sem3))
        if c + 1 < NCC:
            cur1, cur2 = nxt1, nxt2
    stores[NCC - 2].wait()
    stores[NCC - 1].wait()


def _combine(wout, pos1, pos2):
    mesh = plsc.VectorSubcoreMesh(core_axis_name="c", subcore_axis_name="s")
    fn = pl.kernel(
        _combine_sc_body, mesh=mesh,
        out_type=jax.ShapeDtypeStruct((T, HIDDEN), jnp.float32),
        scratch_types=[
            pltpu.VMEM((TOK_PER_W,), jnp.int32),
            pltpu.VMEM((TOK_PER_W,), jnp.int32),
            pltpu.VMEM((2, CCHUNK, HIDDEN), jnp.float32),
            pltpu.VMEM((2, CCHUNK, HIDDEN), jnp.float32),
            pltpu.SemaphoreType.DMA,
            pltpu.SemaphoreType.DMA,
            pltpu.SemaphoreType.DMA,
        ],
    )
    return fn(wout, pos1, pos2)


@jax.jit
def kernel(hidden_states, gate_w, wg, wu, wd):
    B, S, H = hidden_states.shape
    hidden = hidden_states.reshape(-1, H)
    pos, wp, meta = _router(hidden, gate_w)
    pos3 = pos.reshape(NW, NDCHUNK, DCHUNK)
    meta_flat = meta.reshape(NMETA)
    a_sorted, w_sorted = _dispatch(pos3, wp.reshape(NW, NDCHUNK, DCHUNK),
                                   hidden)
    wout = _ffn_gemm(meta_flat, a_sorted, w_sorted.reshape(PADDED, 1),
                     wg, wu, wd)
    final = _combine(wout, pos[0], pos[1])
    return final.reshape(B, S, H)
